# baseline (speedup 1.0000x reference)
import functools

import jax
import jax.numpy as jnp
from jax import lax
from jax.experimental import pallas as pl
from jax.experimental.pallas import tpu as pltpu
from jax.experimental.pallas import tpu_sc as plsc

VOCAB = 100000
EMBD = 128
CTX = 10
HID = 512
NIDX = 2 * CTX

VT = 2048
NT = (VOCAB + VT - 1) // VT
PADV = NT * VT


def _sc_gather(table, idx):
    mesh = plsc.VectorSubcoreMesh(core_axis_name="c", subcore_axis_name="s")

    @functools.partial(
        pl.kernel,
        mesh=mesh,
        out_type=jax.ShapeDtypeStruct((NIDX, EMBD), jnp.float32),
        scratch_types=[
            pltpu.VMEM((NIDX,), jnp.int32),
            pltpu.VMEM((NIDX, EMBD), jnp.float32),
            pltpu.SemaphoreType.DMA,
        ],
    )
    def gather_k(table_hbm, idx_hbm, out_hbm, idx_v, rows_v, sem):
        wid = lax.axis_index("s") * 2 + lax.axis_index("c")

        @pl.when(wid == 0)
        def _():
            pltpu.sync_copy(idx_hbm, idx_v)
            pltpu.async_copy(table_hbm.at[idx_v], rows_v, sem).wait()
            pltpu.sync_copy(rows_v, out_hbm)

    return gather_k(table, idx)


def _hid_body(e_ref, w1_ref, b1_ref, o_ref):
    o_ref[...] = jnp.maximum(
        jnp.dot(e_ref[...], w1_ref[...], preferred_element_type=jnp.float32)
        + b1_ref[...],
        0.0,
    )


def _out_body(hid_ref, w2_ref, b2_ref, o_ref):
    j = pl.program_id(0)
    t = (
        jnp.dot(hid_ref[...], w2_ref[...], preferred_element_type=jnp.float32)
        + b2_ref[...]
    )
    o_ref[:, pl.ds(j * VT, VT)] = t

    @pl.when(j == NT - 1)
    def _():
        col = lax.broadcasted_iota(jnp.int32, (1, PADV), 1)
        full = jnp.where(col < VOCAB, o_ref[...], -jnp.inf)
        m = jnp.max(full)
        s = jnp.sum(jnp.exp(full - m))
        o_ref[...] = full - (m + jnp.log(s))


def kernel(inputs, table, W1, b1, W2, b2):
    idx = inputs.astype(jnp.int32)
    emb = _sc_gather(table, idx)
    emb_flat = emb.reshape(1, NIDX * EMBD)

    hid = pl.pallas_call(
        _hid_body,
        out_shape=jax.ShapeDtypeStruct((1, HID), jnp.float32),
    )(emb_flat, W1, b1.reshape(1, HID))

    log_probs = pl.pallas_call(
        _out_body,
        grid=(NT,),
        in_specs=[
            pl.BlockSpec((1, HID), lambda j: (0, 0)),
            pl.BlockSpec((HID, VT), lambda j: (0, j)),
            pl.BlockSpec((1, VT), lambda j: (0, j)),
        ],
        out_specs=pl.BlockSpec((1, PADV), lambda j: (0, 0)),
        out_shape=jax.ShapeDtypeStruct((1, PADV), jnp.float32),
    )(hid, W2, b2.reshape(1, VOCAB))

    return log_probs[:, :VOCAB]


# speedup vs baseline: 1.0242x; 1.0242x over previous
import functools

import jax
import jax.numpy as jnp
from jax import lax
from jax.experimental import pallas as pl
from jax.experimental.pallas import tpu as pltpu
from jax.experimental.pallas import tpu_sc as plsc

VOCAB = 100000
EMBD = 128
CTX = 10
HID = 512
NIDX = 2 * CTX

K_T = 16
NK = HID // K_T


def _sc_gather(table, idx):
    mesh = plsc.VectorSubcoreMesh(core_axis_name="c", subcore_axis_name="s")

    @functools.partial(
        pl.kernel,
        mesh=mesh,
        out_type=jax.ShapeDtypeStruct((NIDX, EMBD), jnp.float32),
        scratch_types=[
            pltpu.VMEM((NIDX,), jnp.int32),
            pltpu.VMEM((NIDX, EMBD), jnp.float32),
            pltpu.SemaphoreType.DMA,
        ],
    )
    def gather_k(table_hbm, idx_hbm, out_hbm, idx_v, rows_v, sem):
        wid = lax.axis_index("s") * 2 + lax.axis_index("c")

        @pl.when(wid == 0)
        def _():
            pltpu.sync_copy(idx_hbm, idx_v)
            pltpu.async_copy(table_hbm.at[idx_v], rows_v, sem).wait()
            pltpu.sync_copy(rows_v, out_hbm)

    return gather_k(table, idx)


def _hid_body(e_ref, w1_ref, b1_ref, o_ref):
    o_ref[...] = jnp.maximum(
        jnp.dot(e_ref[...], w1_ref[...], preferred_element_type=jnp.float32)
        + b1_ref[...],
        0.0,
    )


def _out_body(hid_ref, w2_ref, b2_ref, o_ref):
    j = pl.program_id(0)
    t = jnp.dot(hid_ref[0], w2_ref[...], preferred_element_type=jnp.float32)

    @pl.when(j == 0)
    def _():
        o_ref[...] = t + b2_ref[...]

    @pl.when(j != 0)
    def _():
        o_ref[...] = o_ref[...] + t

    @pl.when(j == NK - 1)
    def _():
        full = o_ref[...]
        m = jnp.max(full)
        s = jnp.sum(jnp.exp(full - m))
        o_ref[...] = full - (m + jnp.log(s))


def kernel(inputs, table, W1, b1, W2, b2):
    idx = inputs.astype(jnp.int32)
    emb = _sc_gather(table, idx)
    emb_flat = emb.reshape(1, NIDX * EMBD)

    hid = pl.pallas_call(
        _hid_body,
        out_shape=jax.ShapeDtypeStruct((1, HID), jnp.float32),
    )(emb_flat, W1, b1.reshape(1, HID))

    log_probs = pl.pallas_call(
        _out_body,
        grid=(NK,),
        in_specs=[
            pl.BlockSpec((1, 1, K_T), lambda j: (j, 0, 0)),
            pl.BlockSpec((K_T, VOCAB), lambda j: (j, 0)),
            pl.BlockSpec((1, VOCAB), lambda j: (0, 0)),
        ],
        out_specs=pl.BlockSpec((1, VOCAB), lambda j: (0, 0)),
        out_shape=jax.ShapeDtypeStruct((1, VOCAB), jnp.float32),
    )(hid.reshape(NK, 1, K_T), W2, b2.reshape(1, VOCAB))

    return log_probs
